# Initial kernel scaffold; baseline (speedup 1.0000x reference)
import jax
import jax.numpy as jnp
from jax.experimental import pallas as pl


def kernel(x, edge_index, edge_attr, Wq, Wk, We, Wv, Wo, bo, ln1_g, ln1_b, W1, b1, W2, b2, ln2_g, ln2_b):
    raise NotImplementedError("write your pallas kernel here")


# speedup vs baseline: 18.0435x; 18.0435x over previous
import functools

import jax
import jax.numpy as jnp
from jax import lax
from jax.experimental import pallas as pl
from jax.experimental.pallas import tpu as pltpu
from jax.experimental.pallas import tpu_sc as plsc

N = 10000
E = 320000
D = 128
H = 8
DH = 16
NC = 2
NS = 16
NW = NC * NS
EPT = E // NW
CH = 40
NCHUNK = EPT // CH
CH2 = 40
NPAD = 10240
RPT = NPAD // NS

_TC_ROWS = 1000


def _qkv_body(x_ref, wq_ref, wk_ref, wv_ref, q_ref, kv_ref):
    xb = x_ref[...]
    q_ref[...] = jnp.dot(xb, wq_ref[...], preferred_element_type=jnp.float32)
    k = jnp.dot(xb, wk_ref[...], preferred_element_type=jnp.float32)
    v = jnp.dot(xb, wv_ref[...], preferred_element_type=jnp.float32)
    kv_ref[...] = jnp.concatenate([k, v], axis=1)


def _qkv(x, Wq, Wk, Wv):
    nblk = N // _TC_ROWS
    return pl.pallas_call(
        _qkv_body,
        grid=(nblk,),
        in_specs=[
            pl.BlockSpec((_TC_ROWS, D), lambda i: (i, 0)),
            pl.BlockSpec((D, D), lambda i: (0, 0)),
            pl.BlockSpec((D, D), lambda i: (0, 0)),
            pl.BlockSpec((D, D), lambda i: (0, 0)),
        ],
        out_specs=[
            pl.BlockSpec((_TC_ROWS, D), lambda i: (i, 0)),
            pl.BlockSpec((_TC_ROWS, 2 * D), lambda i: (i, 0)),
        ],
        out_shape=[
            jax.ShapeDtypeStruct((N, D), jnp.float32),
            jax.ShapeDtypeStruct((N, 2 * D), jnp.float32),
        ],
    )(x, Wq, Wk, Wv)


def _ep_body(ea_ref, w_ref, out_ref):
    out_ref[...] = jnp.dot(ea_ref[...], w_ref[...],
                           preferred_element_type=jnp.float32)


def _edge_proj(ea2, Wbig):
    rows = E // 8
    nblk = rows // _TC_ROWS
    return pl.pallas_call(
        _ep_body,
        grid=(nblk,),
        in_specs=[
            pl.BlockSpec((_TC_ROWS, D), lambda i: (i, 0)),
            pl.BlockSpec((D, 8 * D), lambda i: (0, 0)),
        ],
        out_specs=pl.BlockSpec((_TC_ROWS, 8 * D), lambda i: (i, 0)),
        out_shape=jax.ShapeDtypeStruct((rows, 8 * D), jnp.float32),
    )(ea2, Wbig)


def _edge_body(kv_hbm, q_hbm, ep_hbm, src_hbm, dst_hbm, zm_hbm,
               outm_hbm, zsc_hbm,
               srcv, dstv, kvb, qb, eb, obm, obz, accm, sem1, sem2):
    c = lax.axis_index("c")
    s = lax.axis_index("s")
    wid = s * NC + c
    r0 = pl.multiple_of(s * RPT, 8)

    pltpu.sync_copy(zm_hbm.at[pl.ds(r0, RPT)], accm.at[pl.ds(r0, RPT)])
    plsc.subcore_barrier()

    iot = lax.iota(jnp.int32, 16)

    def chunk(i, carry):
        base = pl.multiple_of(wid * EPT + i * CH, 8)
        pltpu.sync_copy(src_hbm.at[pl.ds(base, CH)], srcv)
        pltpu.sync_copy(dst_hbm.at[pl.ds(base, CH)], dstv)
        cp1 = pltpu.async_copy(kv_hbm.at[srcv], kvb, sem1)
        cp2 = pltpu.async_copy(q_hbm.at[dstv], qb, sem2)
        pltpu.sync_copy(ep_hbm.at[pl.ds(base, CH)], eb)
        cp1.wait()
        cp2.wait()

        def edge(j, carry2):
            zv = jnp.zeros((16,), jnp.float32)
            for h in range(H):
                k = kvb[j, pl.ds(DH * h, DH)]
                q = qb[j, pl.ds(DH * h, DH)]
                e = eb[j, pl.ds(DH * h, DH)]
                t = k * q * e
                for sh in (8, 4, 2, 1):
                    t = t + jnp.take(t, jnp.bitwise_xor(iot, sh))
                zv = jnp.where(iot == h, t, zv)
            zv = jnp.exp(jnp.clip(zv * 0.25, -5.0, 5.0))
            for h in range(H):
                v = kvb[j, pl.ds(D + DH * h, DH)]
                sb = jnp.take(zv, iot * 0 + h)
                obm[j, pl.ds(DH * h, DH)] = v * sb
            obz[j, :] = zv
            return carry2

        lax.fori_loop(0, CH, edge, 0)
        pltpu.sync_copy(obm, accm.at[dstv], add=True)
        pltpu.sync_copy(obz, zsc_hbm.at[pl.ds(base, CH)])
        return carry

    lax.fori_loop(0, NCHUNK, chunk, 0)
    plsc.subcore_barrier()

    pltpu.sync_copy(accm.at[pl.ds(r0, RPT)], outm_hbm.at[c, pl.ds(r0, RPT)])


def _z_body(zsc_hbm, dst_hbm, zz_hbm, outz_hbm, dstv, zb, ob128, accz, sem1):
    c = lax.axis_index("c")
    s = lax.axis_index("s")
    wid = s * NC + c
    r0 = pl.multiple_of(s * RPT, 8)
    pltpu.sync_copy(zz_hbm.at[pl.ds(r0, RPT)], accz.at[pl.ds(r0, RPT)])

    def zrow(j, carry):
        for t in range(8):
            ob128[j, pl.ds(16 * t, 16)] = jnp.zeros((16,), jnp.float32)
        return carry

    lax.fori_loop(0, CH2, zrow, 0)
    plsc.subcore_barrier()

    def chunk(i, carry):
        base = pl.multiple_of(wid * EPT + i * CH2, 8)
        pltpu.sync_copy(dst_hbm.at[pl.ds(base, CH2)], dstv)
        pltpu.sync_copy(zsc_hbm.at[pl.ds(base, CH2)], zb)

        def mv(j, carry2):
            ob128[j, pl.ds(0, DH)] = zb[j, :]
            return carry2

        lax.fori_loop(0, CH2, mv, 0)
        pltpu.sync_copy(ob128, accz.at[dstv], add=True)
        return carry

    lax.fori_loop(0, EPT // CH2, chunk, 0)
    plsc.subcore_barrier()
    pltpu.sync_copy(accz.at[pl.ds(r0, RPT)], outz_hbm.at[c, pl.ds(r0, RPT)])


def _edge_kernel(kv, q, ep, src, dst, zeros_m):
    mesh = plsc.VectorSubcoreMesh(core_axis_name="c", subcore_axis_name="s",
                                  num_cores=NC, num_subcores=NS)
    fn = pl.kernel(
        _edge_body,
        out_type=(
            jax.ShapeDtypeStruct((NC, NPAD, D), jnp.float32),
            jax.ShapeDtypeStruct((E, DH), jnp.float32),
        ),
        mesh=mesh,
        scratch_types=[
            pltpu.VMEM((CH,), jnp.int32),
            pltpu.VMEM((CH,), jnp.int32),
            pltpu.VMEM((CH, 2 * D), jnp.float32),
            pltpu.VMEM((CH, D), jnp.float32),
            pltpu.VMEM((CH, D), jnp.float32),
            pltpu.VMEM((CH, D), jnp.float32),
            pltpu.VMEM((CH, DH), jnp.float32),
            pltpu.VMEM_SHARED((NPAD, D), jnp.float32),
            pltpu.SemaphoreType.DMA,
            pltpu.SemaphoreType.DMA,
        ],
    )
    return fn(kv, q, ep, src, dst, zeros_m)


def _z_kernel(zsc, dst, zeros_m):
    mesh = plsc.VectorSubcoreMesh(core_axis_name="c", subcore_axis_name="s",
                                  num_cores=NC, num_subcores=NS)
    fn = pl.kernel(
        _z_body,
        out_type=jax.ShapeDtypeStruct((NC, NPAD, D), jnp.float32),
        mesh=mesh,
        scratch_types=[
            pltpu.VMEM((CH2,), jnp.int32),
            pltpu.VMEM((CH2, DH), jnp.float32),
            pltpu.VMEM((CH2, D), jnp.float32),
            pltpu.VMEM_SHARED((NPAD, D), jnp.float32),
            pltpu.SemaphoreType.DMA,
        ],
    )
    return fn(zsc, dst, zeros_m)


def _post_body(pm_ref, pz_ref, x_ref, r16_ref, wo_ref, bo_ref, g1_ref, b1_ref,
               w1_ref, bf1_ref, w2_ref, bf2_ref, g2_ref, b2_ref, out_ref):
    wv = pm_ref[0] + pm_ref[1]
    z = pz_ref[0] + pz_ref[1]
    zx = jnp.dot(z, r16_ref[...], preferred_element_type=jnp.float32)
    ho = wv / (zx + 1e-6)
    h = jnp.dot(ho, wo_ref[...], preferred_element_type=jnp.float32)
    h = h + bo_ref[...] + x_ref[...]
    m = jnp.mean(h, axis=-1, keepdims=True)
    v = jnp.mean((h - m) ** 2, axis=-1, keepdims=True)
    h = (h - m) * lax.rsqrt(v + 1e-5) * g1_ref[...] + b1_ref[...]
    f = jnp.dot(h, w1_ref[...], preferred_element_type=jnp.float32)
    f = jnp.maximum(f + bf1_ref[...], 0.0)
    f = jnp.dot(f, w2_ref[...], preferred_element_type=jnp.float32) + bf2_ref[...]
    h = h + f
    m = jnp.mean(h, axis=-1, keepdims=True)
    v = jnp.mean((h - m) ** 2, axis=-1, keepdims=True)
    out_ref[...] = (h - m) * lax.rsqrt(v + 1e-5) * g2_ref[...] + b2_ref[...]


def _post(pm, pz, x, R16, Wo, bo, ln1_g, ln1_b, W1, b1, W2, b2, ln2_g, ln2_b):
    nblk = N // _TC_ROWS
    vec = lambda d: pl.BlockSpec((d,), lambda i: (0,))
    full = lambda a, b: pl.BlockSpec((a, b), lambda i: (0, 0))
    return pl.pallas_call(
        _post_body,
        grid=(nblk,),
        in_specs=[
            pl.BlockSpec((NC, _TC_ROWS, D), lambda i: (0, i, 0)),
            pl.BlockSpec((NC, _TC_ROWS, D), lambda i: (0, i, 0)),
            pl.BlockSpec((_TC_ROWS, D), lambda i: (i, 0)),
            full(D, D), full(D, D), vec(D), vec(D), vec(D),
            full(D, 2 * D), vec(2 * D), full(2 * D, D), vec(D), vec(D), vec(D),
        ],
        out_specs=pl.BlockSpec((_TC_ROWS, D), lambda i: (i, 0)),
        out_shape=jax.ShapeDtypeStruct((N, D), jnp.float32),
    )(pm, pz, x, R16, Wo, bo, ln1_g, ln1_b, W1, b1, W2, b2, ln2_g, ln2_b)


def kernel(x, edge_index, edge_attr, Wq, Wk, We, Wv, Wo, bo, ln1_g, ln1_b,
           W1, b1, W2, b2, ln2_g, ln2_b):
    src = edge_index[0]
    dst = edge_index[1]

    q, kv = _qkv(x, Wq, Wk, Wv)

    Wbig = jnp.kron(jnp.eye(8, dtype=jnp.float32), We)
    ep = _edge_proj(edge_attr.reshape(E // 8, 8 * DH), Wbig).reshape(E, D)

    zeros_m = jnp.zeros((NPAD, D), jnp.float32)
    pm, zsc = _edge_kernel(kv, q, ep, src, dst, zeros_m)
    pz = _z_kernel(zsc, dst, zeros_m)

    R16 = jnp.kron(jnp.eye(8, dtype=jnp.float32), jnp.ones((1, DH), jnp.float32))
    R16 = jnp.concatenate([R16, jnp.zeros((D - 8, D), jnp.float32)], axis=0)

    return _post(pm, pz, x, R16, Wo, bo, ln1_g, ln1_b, W1, b1, W2, b2,
                 ln2_g, ln2_b)


# speedup vs baseline: 18.9243x; 1.0488x over previous
import functools

import jax
import jax.numpy as jnp
from jax import lax
from jax.experimental import pallas as pl
from jax.experimental.pallas import tpu as pltpu
from jax.experimental.pallas import tpu_sc as plsc

N = 10000
E = 320000
D = 128
H = 8
DH = 16
NC = 2
NS = 16
NW = NC * NS
EPT = E // NW
CH = 40
NCHUNK = EPT // CH
CH2 = 80
NPAD = 10240
RPT = NPAD // NS

_TC_ROWS = 1000


def _qkv_body(x_ref, wq_ref, wk_ref, wv_ref, q_ref, kv_ref):
    xb = x_ref[...]
    q_ref[...] = jnp.dot(xb, wq_ref[...], preferred_element_type=jnp.float32)
    k = jnp.dot(xb, wk_ref[...], preferred_element_type=jnp.float32)
    v = jnp.dot(xb, wv_ref[...], preferred_element_type=jnp.float32)
    kv_ref[...] = jnp.concatenate([k, v], axis=1)


def _qkv(x, Wq, Wk, Wv):
    nblk = N // _TC_ROWS
    return pl.pallas_call(
        _qkv_body,
        grid=(nblk,),
        in_specs=[
            pl.BlockSpec((_TC_ROWS, D), lambda i: (i, 0)),
            pl.BlockSpec((D, D), lambda i: (0, 0)),
            pl.BlockSpec((D, D), lambda i: (0, 0)),
            pl.BlockSpec((D, D), lambda i: (0, 0)),
        ],
        out_specs=[
            pl.BlockSpec((_TC_ROWS, D), lambda i: (i, 0)),
            pl.BlockSpec((_TC_ROWS, 2 * D), lambda i: (i, 0)),
        ],
        out_shape=[
            jax.ShapeDtypeStruct((N, D), jnp.float32),
            jax.ShapeDtypeStruct((N, 2 * D), jnp.float32),
        ],
    )(x, Wq, Wk, Wv)


def _ep_body(ea_ref, w_ref, out_ref):
    out_ref[...] = jnp.dot(ea_ref[...], w_ref[...],
                           preferred_element_type=jnp.float32)


def _edge_proj(ea2, Wbig):
    rows = E // 8
    nblk = rows // _TC_ROWS
    return pl.pallas_call(
        _ep_body,
        grid=(nblk,),
        in_specs=[
            pl.BlockSpec((_TC_ROWS, D), lambda i: (i, 0)),
            pl.BlockSpec((D, 8 * D), lambda i: (0, 0)),
        ],
        out_specs=pl.BlockSpec((_TC_ROWS, 8 * D), lambda i: (i, 0)),
        out_shape=jax.ShapeDtypeStruct((rows, 8 * D), jnp.float32),
    )(ea2, Wbig)


def _edge_body(kv_hbm, q_hbm, ep_hbm, src_hbm, dst_hbm, zm_hbm,
               outm_hbm, zsc_hbm,
               srcv, dstv, kvb, qb, eb, obm, obz, accm, sem1, sem2):
    c = lax.axis_index("c")
    s = lax.axis_index("s")
    wid = s * NC + c
    r0 = pl.multiple_of(s * RPT, 8)

    pltpu.sync_copy(zm_hbm.at[pl.ds(r0, RPT)], accm.at[pl.ds(r0, RPT)])
    plsc.subcore_barrier()

    iot = lax.iota(jnp.int32, 16)
    pa2 = jnp.bitwise_and(jnp.where(iot < 4, iot, iot + 4), 15)
    pb2 = jnp.where(iot < 12, jnp.bitwise_and(iot - 8, 15),
                    jnp.bitwise_and(iot - 4, 15))
    pa3 = jnp.bitwise_and(((iot >> 1) << 2) | (iot & 1), 15)
    pb3 = jnp.bitwise_and((((iot - 8) >> 1) << 2) | (iot & 1), 15)
    pe = (iot & 7) * 2

    def chunk(i, carry):
        base = pl.multiple_of(wid * EPT + i * CH, 8)
        pltpu.sync_copy(src_hbm.at[pl.ds(base, CH)], srcv)
        pltpu.sync_copy(dst_hbm.at[pl.ds(base, CH)], dstv)
        cp1 = pltpu.async_copy(kv_hbm.at[srcv], kvb, sem1)
        cp2 = pltpu.async_copy(q_hbm.at[dstv], qb, sem2)
        pltpu.sync_copy(ep_hbm.at[pl.ds(base, CH)], eb)
        cp1.wait()
        cp2.wait()

        def edge(j, carry2):
            t = []
            for h in range(H):
                k = kvb[j, pl.ds(DH * h, DH)]
                q = qb[j, pl.ds(DH * h, DH)]
                e = eb[j, pl.ds(DH * h, DH)]
                t.append(k * q * e)
            u = []
            for p in range(4):
                a = t[2 * p] + jnp.take(t[2 * p], jnp.bitwise_xor(iot, 8))
                b = t[2 * p + 1] + jnp.take(t[2 * p + 1], jnp.bitwise_xor(iot, 8))
                u.append(jnp.where(iot < 8, a, b))
            v2 = []
            for p in range(2):
                a = u[2 * p] + jnp.take(u[2 * p], jnp.bitwise_xor(iot, 4))
                b = u[2 * p + 1] + jnp.take(u[2 * p + 1], jnp.bitwise_xor(iot, 4))
                v2.append(jnp.where(iot < 8, jnp.take(a, pa2), jnp.take(b, pb2)))
            a = v2[0] + jnp.take(v2[0], jnp.bitwise_xor(iot, 2))
            b = v2[1] + jnp.take(v2[1], jnp.bitwise_xor(iot, 2))
            w = jnp.where(iot < 8, jnp.take(a, pa3), jnp.take(b, pb3))
            w = w + jnp.take(w, jnp.bitwise_xor(iot, 1))
            zv = jnp.take(w, pe)
            zv = jnp.exp(jnp.clip(zv * 0.25, -5.0, 5.0))
            for h in range(H):
                v = kvb[j, pl.ds(D + DH * h, DH)]
                sb = jnp.take(zv, iot * 0 + h)
                obm[j, pl.ds(DH * h, DH)] = v * sb
            obz[j, :] = zv
            return carry2

        lax.fori_loop(0, CH, edge, 0)
        pltpu.sync_copy(obm, accm.at[dstv], add=True)
        pltpu.sync_copy(obz, zsc_hbm.at[pl.ds(base, CH)])
        return carry

    lax.fori_loop(0, NCHUNK, chunk, 0)
    plsc.subcore_barrier()

    pltpu.sync_copy(accm.at[pl.ds(r0, RPT)], outm_hbm.at[c, pl.ds(r0, RPT)])


def _z_body(zsc_hbm, dst_hbm, zz_hbm, outz_hbm, dstv, zb, ob128, accz, sem1):
    c = lax.axis_index("c")
    s = lax.axis_index("s")
    wid = s * NC + c
    r0 = pl.multiple_of(s * RPT, 8)
    pltpu.sync_copy(zz_hbm.at[pl.ds(r0, RPT)], accz.at[pl.ds(r0, RPT)])

    def zrow(j, carry):
        for t in range(8):
            ob128[j, pl.ds(16 * t, 16)] = jnp.zeros((16,), jnp.float32)
        return carry

    lax.fori_loop(0, CH2, zrow, 0)
    plsc.subcore_barrier()

    def chunk(i, carry):
        base = pl.multiple_of(wid * EPT + i * CH2, 8)
        pltpu.sync_copy(dst_hbm.at[pl.ds(base, CH2)], dstv)
        pltpu.sync_copy(zsc_hbm.at[pl.ds(base, CH2)], zb)

        def mv(j, carry2):
            ob128[j, pl.ds(0, DH)] = zb[j, :]
            return carry2

        lax.fori_loop(0, CH2, mv, 0)
        pltpu.sync_copy(ob128, accz.at[dstv], add=True)
        return carry

    lax.fori_loop(0, EPT // CH2, chunk, 0)
    plsc.subcore_barrier()
    pltpu.sync_copy(accz.at[pl.ds(r0, RPT)], outz_hbm.at[c, pl.ds(r0, RPT)])


def _edge_kernel(kv, q, ep, src, dst, zeros_m):
    mesh = plsc.VectorSubcoreMesh(core_axis_name="c", subcore_axis_name="s",
                                  num_cores=NC, num_subcores=NS)
    fn = pl.kernel(
        _edge_body,
        out_type=(
            jax.ShapeDtypeStruct((NC, NPAD, D), jnp.float32),
            jax.ShapeDtypeStruct((E, DH), jnp.float32),
        ),
        mesh=mesh,
        scratch_types=[
            pltpu.VMEM((CH,), jnp.int32),
            pltpu.VMEM((CH,), jnp.int32),
            pltpu.VMEM((CH, 2 * D), jnp.float32),
            pltpu.VMEM((CH, D), jnp.float32),
            pltpu.VMEM((CH, D), jnp.float32),
            pltpu.VMEM((CH, D), jnp.float32),
            pltpu.VMEM((CH, DH), jnp.float32),
            pltpu.VMEM_SHARED((NPAD, D), jnp.float32),
            pltpu.SemaphoreType.DMA,
            pltpu.SemaphoreType.DMA,
        ],
    )
    return fn(kv, q, ep, src, dst, zeros_m)


def _z_kernel(zsc, dst, zeros_m):
    mesh = plsc.VectorSubcoreMesh(core_axis_name="c", subcore_axis_name="s",
                                  num_cores=NC, num_subcores=NS)
    fn = pl.kernel(
        _z_body,
        out_type=jax.ShapeDtypeStruct((NC, NPAD, D), jnp.float32),
        mesh=mesh,
        scratch_types=[
            pltpu.VMEM((CH2,), jnp.int32),
            pltpu.VMEM((CH2, DH), jnp.float32),
            pltpu.VMEM((CH2, D), jnp.float32),
            pltpu.VMEM_SHARED((NPAD, D), jnp.float32),
            pltpu.SemaphoreType.DMA,
        ],
    )
    return fn(zsc, dst, zeros_m)


def _post_body(pm_ref, pz_ref, x_ref, r16_ref, wo_ref, bo_ref, g1_ref, b1_ref,
               w1_ref, bf1_ref, w2_ref, bf2_ref, g2_ref, b2_ref, out_ref):
    wv = pm_ref[0] + pm_ref[1]
    z = pz_ref[0] + pz_ref[1]
    zx = jnp.dot(z, r16_ref[...], preferred_element_type=jnp.float32)
    ho = wv / (zx + 1e-6)
    h = jnp.dot(ho, wo_ref[...], preferred_element_type=jnp.float32)
    h = h + bo_ref[...] + x_ref[...]
    m = jnp.mean(h, axis=-1, keepdims=True)
    v = jnp.mean((h - m) ** 2, axis=-1, keepdims=True)
    h = (h - m) * lax.rsqrt(v + 1e-5) * g1_ref[...] + b1_ref[...]
    f = jnp.dot(h, w1_ref[...], preferred_element_type=jnp.float32)
    f = jnp.maximum(f + bf1_ref[...], 0.0)
    f = jnp.dot(f, w2_ref[...], preferred_element_type=jnp.float32) + bf2_ref[...]
    h = h + f
    m = jnp.mean(h, axis=-1, keepdims=True)
    v = jnp.mean((h - m) ** 2, axis=-1, keepdims=True)
    out_ref[...] = (h - m) * lax.rsqrt(v + 1e-5) * g2_ref[...] + b2_ref[...]


def _post(pm, pz, x, R16, Wo, bo, ln1_g, ln1_b, W1, b1, W2, b2, ln2_g, ln2_b):
    nblk = N // _TC_ROWS
    vec = lambda d: pl.BlockSpec((d,), lambda i: (0,))
    full = lambda a, b: pl.BlockSpec((a, b), lambda i: (0, 0))
    return pl.pallas_call(
        _post_body,
        grid=(nblk,),
        in_specs=[
            pl.BlockSpec((NC, _TC_ROWS, D), lambda i: (0, i, 0)),
            pl.BlockSpec((NC, _TC_ROWS, D), lambda i: (0, i, 0)),
            pl.BlockSpec((_TC_ROWS, D), lambda i: (i, 0)),
            full(D, D), full(D, D), vec(D), vec(D), vec(D),
            full(D, 2 * D), vec(2 * D), full(2 * D, D), vec(D), vec(D), vec(D),
        ],
        out_specs=pl.BlockSpec((_TC_ROWS, D), lambda i: (i, 0)),
        out_shape=jax.ShapeDtypeStruct((N, D), jnp.float32),
    )(pm, pz, x, R16, Wo, bo, ln1_g, ln1_b, W1, b1, W2, b2, ln2_g, ln2_b)


def kernel(x, edge_index, edge_attr, Wq, Wk, We, Wv, Wo, bo, ln1_g, ln1_b,
           W1, b1, W2, b2, ln2_g, ln2_b):
    src = edge_index[0]
    dst = edge_index[1]

    q, kv = _qkv(x, Wq, Wk, Wv)

    Wbig = jnp.kron(jnp.eye(8, dtype=jnp.float32), We)
    ep = _edge_proj(edge_attr.reshape(E // 8, 8 * DH), Wbig).reshape(E, D)

    zeros_m = jnp.zeros((NPAD, D), jnp.float32)
    pm, zsc = _edge_kernel(kv, q, ep, src, dst, zeros_m)
    pz = _z_kernel(zsc, dst, zeros_m)

    R16 = jnp.kron(jnp.eye(8, dtype=jnp.float32), jnp.ones((1, DH), jnp.float32))
    R16 = jnp.concatenate([R16, jnp.zeros((D - 8, D), jnp.float32)], axis=0)

    return _post(pm, pz, x, R16, Wo, bo, ln1_g, ln1_b, W1, b1, W2, b2,
                 ln2_g, ln2_b)
